# baseline (speedup 1.0000x reference)
import jax
import jax.numpy as jnp
from jax.experimental import pallas as pl
from jax.experimental.pallas import tpu as pltpu

EMBED = 1024
NUM_HEADS = 16
HEAD_DIM = EMBED // NUM_HEADS
NUM_BLOCKS = 64
BS = 64
N_RAND = 3
SCALING = HEAD_DIM ** (-0.5)
NEG_INF = -1e30


def _proj_body(x_ref, w_ref, b_ref, o_ref):
    acc = jnp.dot(x_ref[...], w_ref[...], preferred_element_type=jnp.float32)
    o_ref[...] = (acc + b_ref[...]).astype(jnp.bfloat16)


def _qkv_proj(x2d, w_all, b_all, block_m=512):
    m = x2d.shape[0]
    n = w_all.shape[1]
    k = x2d.shape[1]
    return pl.pallas_call(
        _proj_body,
        grid=(m // block_m,),
        in_specs=[
            pl.BlockSpec((block_m, k), lambda i: (i, 0)),
            pl.BlockSpec((k, n), lambda i: (0, 0)),
            pl.BlockSpec((1, n), lambda i: (0, 0)),
        ],
        out_specs=pl.BlockSpec((block_m, n), lambda i: (i, 0)),
        out_shape=jax.ShapeDtypeStruct((m, n), jnp.bfloat16),
    )(x2d, w_all, b_all)


LANES = 2 * HEAD_DIM
G = 4


def _attn_body(rand_ref, *refs):
    nr = 2 * G * N_RAND
    q_ref = refs[0]
    kc, khl, khr = refs[1], refs[2], refs[3]
    kr = refs[4:4 + nr]
    vc, vhl, vhr = refs[4 + nr], refs[5 + nr], refs[6 + nr]
    vr = refs[7 + nr:7 + 2 * nr]
    o_ref = refs[7 + 2 * nr]

    g = pl.program_id(2)
    first_grp = g == 0
    last_grp = g == (NUM_BLOCKS // G - 1)

    lane = jax.lax.broadcasted_iota(jnp.int32, (BS, LANES), 1)
    hmasks = ((lane < HEAD_DIM).astype(jnp.bfloat16),
              (lane >= HEAD_DIM).astype(jnp.bfloat16))

    for g0 in range(G):
        q_pair = q_ref[g0]
        kwin = [khl[0] if g0 == 0 else kc[g0 - 1],
                kc[g0],
                khr[0] if g0 == G - 1 else kc[g0 + 1]]
        vwin = [vhl[0] if g0 == 0 else vc[g0 - 1],
                vc[g0],
                vhr[0] if g0 == G - 1 else vc[g0 + 1]]

        ctxs = []
        for half in range(2):
            j = (half * G + g0) * N_RAND
            k6 = jnp.concatenate(
                kwin + [kr[j][0], kr[j + 1][0], kr[j + 2][0]],
                axis=0)
            scores = jax.lax.dot_general(
                q_pair * hmasks[half], k6, (((1,), (1,)), ((), ())),
                preferred_element_type=jnp.float32)

            if g0 == 0:
                col = jax.lax.broadcasted_iota(jnp.int32, scores.shape, 1)
                scores = jnp.where(first_grp & (col < BS), NEG_INF, scores)
            if g0 == G - 1:
                col = jax.lax.broadcasted_iota(jnp.int32, scores.shape, 1)
                scores = jnp.where(last_grp & (col >= 2 * BS) & (col < 3 * BS),
                                   NEG_INF, scores)

            p = jax.nn.softmax(scores, axis=-1)

            v6 = jnp.concatenate(
                vwin + [vr[j][0], vr[j + 1][0], vr[j + 2][0]],
                axis=0)
            ctxs.append(jnp.dot(p.astype(jnp.bfloat16), v6,
                                preferred_element_type=jnp.float32))

        o_ref[g0] = jnp.where(lane < HEAD_DIM, ctxs[0], ctxs[1])


def _block_attn(qkv, rand_attn, bsz, seqlen):
    n_pairs = NUM_HEADS // 2
    n_grps = NUM_BLOCKS // G

    def grp_spec(grp_off):
        return pl.BlockSpec(
            (None, G, BS, LANES),
            lambda b, a, g, rand_ref: (b, g, 0, grp_off + a))

    def halo_spec(grp_off, right):
        def idx(b, a, g, rand_ref):
            if right:
                return (b, jnp.minimum(g * G + G, NUM_BLOCKS - 1), 0,
                        grp_off + a)
            return (b, jnp.maximum(g * G - 1, 0), 0, grp_off + a)
        return pl.BlockSpec((None, 1, BS, LANES), idx)

    def rand_spec(grp_off, half, g0, r):
        def idx(b, a, g, rand_ref):
            head = 2 * a + half
            blk = g * G + g0
            flat = ((b * NUM_HEADS + head) * NUM_BLOCKS + blk) * N_RAND + r
            return (b, rand_ref[flat], 0, grp_off + a)
        return pl.BlockSpec((None, 1, BS, LANES), idx)

    def side(grp_off):
        return ([grp_spec(grp_off), halo_spec(grp_off, False),
                 halo_spec(grp_off, True)]
                + [rand_spec(grp_off, h, q, r)
                   for h in range(2) for q in range(G) for r in range(N_RAND)])

    in_specs = [grp_spec(0)] + side(n_pairs) + side(2 * n_pairs)

    grid_spec = pltpu.PrefetchScalarGridSpec(
        num_scalar_prefetch=1,
        grid=(bsz, n_pairs, n_grps),
        in_specs=in_specs,
        out_specs=pl.BlockSpec((None, G, BS, LANES),
                               lambda b, a, g, rand_ref: (b, g, 0, a)),
    )
    n_in = len(in_specs)
    return pl.pallas_call(
        _attn_body,
        grid_spec=grid_spec,
        out_shape=jax.ShapeDtypeStruct(
            (bsz, NUM_BLOCKS, BS, NUM_HEADS * HEAD_DIM), jnp.float32),
        compiler_params=pltpu.CompilerParams(
            dimension_semantics=("parallel", "parallel", "arbitrary")),
    )(rand_attn.reshape(-1), *([qkv] * n_in))


def kernel(hidden_states, rand_attn, Wq, bq, Wk, bk, Wv, bv):
    bsz, seqlen, embed = hidden_states.shape

    w_all = jnp.concatenate([Wq.T * SCALING, Wk.T, Wv.T], axis=1)
    b_all = jnp.concatenate([bq * SCALING, bk, bv]).reshape(1, 3 * embed)

    x2d = hidden_states.reshape(bsz * seqlen, embed).astype(jnp.bfloat16)
    qkv = _qkv_proj(x2d, w_all.astype(jnp.bfloat16), b_all)
    qkv = qkv.reshape(bsz, NUM_BLOCKS, BS, 3 * embed)

    ctx = _block_attn(qkv, rand_attn.astype(jnp.int32), bsz, seqlen)
    return ctx.reshape(bsz, seqlen, embed)


# speedup vs baseline: 1.0837x; 1.0837x over previous
import jax
import jax.numpy as jnp
from jax.experimental import pallas as pl
from jax.experimental.pallas import tpu as pltpu

EMBED = 1024
NUM_HEADS = 16
HEAD_DIM = EMBED // NUM_HEADS
NUM_BLOCKS = 64
BS = 64
N_RAND = 3
SCALING = HEAD_DIM ** (-0.5)
NEG_INF = -1e30
LANES = 2 * HEAD_DIM
N_PAIRS = NUM_HEADS // 2
N_COLS = 3 * EMBED // LANES
G = 8


PROJ_BM = 512


def _proj_body(x_ref, w_ref, b_ref, o_ref):
    acc = jnp.dot(x_ref[...], w_ref[...],
                  preferred_element_type=jnp.float32) + b_ref[...]
    t = acc.astype(jnp.bfloat16)
    for c in range(N_COLS):
        for j in range(PROJ_BM // BS):
            o_ref[c, j] = t[j * BS:(j + 1) * BS, c * LANES:(c + 1) * LANES]


def _qkv_proj(x2d, w_all, b_all):
    m, k = x2d.shape
    n = w_all.shape[1]
    bsz = m // (NUM_BLOCKS * BS)
    gb = PROJ_BM // BS
    steps_per_b = NUM_BLOCKS // gb
    return pl.pallas_call(
        _proj_body,
        grid=(m // PROJ_BM,),
        in_specs=[
            pl.BlockSpec((PROJ_BM, k), lambda i: (i, 0)),
            pl.BlockSpec((k, n), lambda i: (0, 0)),
            pl.BlockSpec((1, n), lambda i: (0, 0)),
        ],
        out_specs=pl.BlockSpec(
            (None, N_COLS, gb, BS, LANES),
            lambda i: (i // steps_per_b, 0, i % steps_per_b, 0, 0)),
        out_shape=jax.ShapeDtypeStruct(
            (bsz, N_COLS, NUM_BLOCKS, BS, LANES), jnp.bfloat16),
    )(x2d, w_all, b_all)


def _attn_body(rand_ref, *refs):
    nr = 2 * G * N_RAND
    q_ref, kvc, kvl, kvr = refs[0], refs[1], refs[2], refs[3]
    kv_rand = refs[4:4 + nr]
    o_ref = refs[4 + nr]

    g = pl.program_id(2)
    first_grp = g == 0
    last_grp = g == (NUM_BLOCKS // G - 1)

    lane = jax.lax.broadcasted_iota(jnp.int32, (BS, LANES), 1)
    hmasks = ((lane < HEAD_DIM).astype(jnp.bfloat16),
              (lane >= HEAD_DIM).astype(jnp.bfloat16))

    for g0 in range(G):
        q_pair = q_ref[g0]
        kwin = [kvl[0, 0] if g0 == 0 else kvc[0, g0 - 1],
                kvc[0, g0],
                kvr[0, 0] if g0 == G - 1 else kvc[0, g0 + 1]]
        vwin = [kvl[1, 0] if g0 == 0 else kvc[1, g0 - 1],
                kvc[1, g0],
                kvr[1, 0] if g0 == G - 1 else kvc[1, g0 + 1]]

        ctxs = []
        for half in range(2):
            j = (half * G + g0) * N_RAND
            k6 = jnp.concatenate(
                kwin + [kv_rand[j][0, 0], kv_rand[j + 1][0, 0],
                        kv_rand[j + 2][0, 0]], axis=0)
            scores = jax.lax.dot_general(
                q_pair * hmasks[half], k6, (((1,), (1,)), ((), ())),
                preferred_element_type=jnp.float32)

            if g0 == 0:
                col = jax.lax.broadcasted_iota(jnp.int32, scores.shape, 1)
                scores = jnp.where(first_grp & (col < BS), NEG_INF, scores)
            if g0 == G - 1:
                col = jax.lax.broadcasted_iota(jnp.int32, scores.shape, 1)
                scores = jnp.where(last_grp & (col >= 2 * BS) & (col < 3 * BS),
                                   NEG_INF, scores)

            p = jax.nn.softmax(scores, axis=-1)

            v6 = jnp.concatenate(
                vwin + [kv_rand[j][1, 0], kv_rand[j + 1][1, 0],
                        kv_rand[j + 2][1, 0]], axis=0)
            ctxs.append(jnp.dot(p.astype(jnp.bfloat16), v6,
                                preferred_element_type=jnp.float32))

        o_ref[g0] = jnp.where(lane < HEAD_DIM, ctxs[0], ctxs[1])


def _block_attn(qkv6, rand_attn, bsz):
    n_grps = NUM_BLOCKS // G

    q_spec = pl.BlockSpec((None, None, G, BS, LANES),
                          lambda b, a, g, rand_ref: (b, a, g, 0, 0))

    kv_center = pl.BlockSpec((None, 2, G, BS, LANES),
                             lambda b, a, g, rand_ref: (b, 4 + a, g, 0, 0))

    def kv_halo(right):
        def idx(b, a, g, rand_ref):
            if right:
                return (b, 4 + a,
                        jnp.minimum(g * G + G, NUM_BLOCKS - 1), 0, 0)
            return (b, 4 + a, jnp.maximum(g * G - 1, 0), 0, 0)
        return pl.BlockSpec((None, 2, 1, BS, LANES), idx)

    def kv_rand(half, g0, r):
        def idx(b, a, g, rand_ref):
            head = 2 * a + half
            blk = g * G + g0
            flat = ((b * NUM_HEADS + head) * NUM_BLOCKS + blk) * N_RAND + r
            return (b, 4 + a, rand_ref[flat], 0, 0)
        return pl.BlockSpec((None, 2, 1, BS, LANES), idx)

    in_specs = ([q_spec, kv_center, kv_halo(False), kv_halo(True)]
                + [kv_rand(h, q, r)
                   for h in range(2) for q in range(G) for r in range(N_RAND)])

    grid_spec = pltpu.PrefetchScalarGridSpec(
        num_scalar_prefetch=1,
        grid=(bsz, N_PAIRS, n_grps),
        in_specs=in_specs,
        out_specs=pl.BlockSpec((None, G, BS, LANES),
                               lambda b, a, g, rand_ref: (b, g, 0, a)),
    )
    return pl.pallas_call(
        _attn_body,
        grid_spec=grid_spec,
        out_shape=jax.ShapeDtypeStruct(
            (bsz, NUM_BLOCKS, BS, NUM_HEADS * HEAD_DIM), jnp.float32),
        compiler_params=pltpu.CompilerParams(
            dimension_semantics=("parallel", "parallel", "arbitrary")),
    )(rand_attn.reshape(-1), *([qkv6] * len(in_specs)))


def kernel(hidden_states, rand_attn, Wq, bq, Wk, bk, Wv, bv):
    bsz, seqlen, embed = hidden_states.shape

    wq2 = Wq.T * SCALING
    kv_w = jnp.stack([Wk.T.reshape(embed, N_PAIRS, LANES),
                      Wv.T.reshape(embed, N_PAIRS, LANES)],
                     axis=2).reshape(embed, 2 * embed)
    w_all = jnp.concatenate([wq2, kv_w], axis=1)
    kv_b = jnp.stack([bk.reshape(N_PAIRS, LANES),
                      bv.reshape(N_PAIRS, LANES)], axis=1).reshape(-1)
    b_all = jnp.concatenate([bq * SCALING, kv_b]).reshape(1, 3 * embed)

    x2d = hidden_states.reshape(bsz * seqlen, embed).astype(jnp.bfloat16)
    qkv6 = _qkv_proj(x2d, w_all.astype(jnp.bfloat16), b_all)

    ctx = _block_attn(qkv6, rand_attn.astype(jnp.int32), bsz)
    return ctx.reshape(bsz, seqlen, embed)
